# baseline (speedup 1.0000x reference)
import jax
import jax.numpy as jnp
from jax.experimental import pallas as pl
from jax.experimental.pallas import tpu as pltpu

_C = 80
_G = 20
_K = 10
_CB = 4224
_NC = 8
_AP = _CB * _NC
_LOG_EPS = -27.631021


def _dotg(a, b, dims):
    return jax.lax.dot_general(
        a, b, (dims, ((), ())),
        precision=jax.lax.Precision.HIGHEST,
        preferred_element_type=jnp.float32)


def _yolox_kernel(xs_ref, ys_ref, ss_ref, raw_ref, lab_ref, out_ref,
                  ot5_s, cost_s, iou_s, craw_s, sps_s,
                  cand_iou_s, cand_cv_s, cand_ci_s, sel_s, acc_s,
                  *, n_valid):
    f32 = jnp.float32
    p = pl.program_id(1)
    c = pl.program_id(2)
    inf = jnp.float32(jnp.inf)
    big_i = jnp.int32(2 ** 30)
    li = jax.lax.broadcasted_iota(jnp.int32, (1, _CB), 1)
    l128 = jax.lax.broadcasted_iota(jnp.int32, (1, 128), 1)
    sl = pl.ds(c * _CB, _CB)

    xs = xs_ref[...]
    ys = ys_ref[...]
    ss = ss_ref[...]
    lb = lab_ref[...][0]
    tbx, tby, tbw, tbh = lb[:, 0:1], lb[:, 1:2], lb[:, 2:3], lb[:, 3:4]

    @pl.when(jnp.logical_and(p == 0, c == 0))
    def _init():
        cand_iou_s[...] = jnp.full((_G, 128), -2.0, f32)
        cand_cv_s[...] = jnp.full((_G, 128), inf, f32)
        cand_ci_s[...] = jnp.zeros((_G, 128), f32)
        acc_s[0] = 0.0
        acc_s[1] = 0.0
        acc_s[2] = 0.0
        acc_s[3] = 0.0

    @pl.when(p == 0)
    def _phase_a():
        valid = (li + c * _CB) < n_valid
        raw = raw_ref[...][0]
        vcol = (jax.lax.broadcasted_iota(jnp.int32, (_CB, 1), 0) + c * _CB) < n_valid
        xcls = raw[:, 5:]
        oo_c = jnp.where(vcol, raw[:, 4:5], -1e9)

        eoc = jnp.exp(-jnp.abs(oo_c))
        loc = jnp.log1p(eoc)
        lsig_oc = jnp.minimum(oo_c, 0.0) - loc
        e = jnp.exp(-jnp.abs(xcls))
        l = jnp.log1p(e)
        a = 0.5 * ((jnp.minimum(xcls, 0.0) - l) + lsig_oc)
        score = jnp.exp(a)
        l2 = jnp.maximum(jnp.log1p(1e-12 - score), _LOG_EPS)
        sp = jnp.maximum(xcls, 0.0) + l
        acc_s[1] = acc_s[1] + jnp.sum(jnp.maximum(oo_c, 0.0) + loc)

        e5 = (jax.lax.broadcasted_iota(jnp.int32, (5, 5 + _C), 1) ==
              jax.lax.broadcasted_iota(jnp.int32, (5, 5 + _C), 0)).astype(f32)
        oh = (lb[:, 4:5].astype(jnp.int32) ==
              jax.lax.broadcasted_iota(jnp.int32, (_G, _C), 1)).astype(f32)
        tr5 = _dotg(e5, raw, (((1,), (1,))))
        s_all = _dotg(jnp.ones((1, _C), f32), l2, (((1,), (1,))))
        craw = _dotg(oh, xcls, (((1,), (1,))))
        sps_row = _dotg(jnp.ones((1, _C), f32), sp, (((1,), (1,))))

        box = jnp.where(valid, tr5[0:4], 0.0)
        oo = jnp.where(valid, tr5[4:5], -1e9)
        ot5_s[:, sl] = jnp.concatenate([box, oo], axis=0)

        px = (box[0:1, :] + xs) * ss
        py = (box[1:2, :] + ys) * ss
        pw = jnp.exp(box[2:3, :]) * ss
        ph = jnp.exp(box[3:4, :]) * ss

        xc = (xs + 0.5) * ss
        yc = (ys + 0.5) * ss
        in_box = (jnp.minimum(jnp.minimum(xc - (tbx - tbw / 2), yc - (tby - tbh / 2)),
                              jnp.minimum((tbx + tbw / 2) - xc, (tby + tbh / 2) - yc)) > 0.0)
        rss = 2.5 * ss
        in_mir = (jnp.minimum(jnp.minimum(xc - (tbx - rss), yc - (tby - rss)),
                              jnp.minimum((tbx + rss) - xc, (tby + rss) - yc)) > 0.0)
        union = (jnp.any(in_box, axis=0, keepdims=True) |
                 jnp.any(in_mir, axis=0, keepdims=True))
        inter_m = in_box & in_mir

        tlx = jnp.maximum(tbx - tbw / 2, px - pw / 2)
        tly = jnp.maximum(tby - tbh / 2, py - ph / 2)
        brx = jnp.minimum(tbx + tbw / 2, px + pw / 2)
        bry = jnp.minimum(tby + tbh / 2, py + ph / 2)
        inter = jnp.maximum(brx - tlx, 0.0) * jnp.maximum(bry - tly, 0.0)
        iou = inter / (tbw * tbh + pw * ph - inter + 1e-16)
        iou_s[:, sl] = iou

        sps_s[:, sl] = jnp.where(valid, sps_row, 0.0)
        craw = jnp.where(valid, craw, 0.0)
        craw_s[:, sl] = craw
        lo_r = jnp.log1p(jnp.exp(-jnp.abs(oo)))
        lsig_o = jnp.minimum(oo, 0.0) - lo_r
        lt = jnp.log1p(jnp.exp(-jnp.abs(craw)))
        at = 0.5 * ((jnp.minimum(craw, 0.0) - lt) + lsig_o)
        g1 = jnp.maximum(at, _LOG_EPS)
        g2 = jnp.maximum(jnp.log1p(1e-12 - jnp.exp(at)), _LOG_EPS)

        cls_loss = -(s_all + g1 - g2)
        iou_loss = -jnp.log(iou + 1e-8)
        cost = cls_loss + 3.0 * iou_loss + 100000.0 * (1.0 - inter_m.astype(f32))
        cost = jnp.where(union, cost, inf)
        cost_s[:, sl] = cost

        ws = jnp.concatenate([-jnp.where(union, iou, -1.0), cost], axis=0)
        cand_iou = cand_iou_s[...]
        cand_cv = cand_cv_s[...]
        cand_ci = cand_ci_s[...]
        for k in range(_K):
            m = jnp.min(ws, axis=1, keepdims=True)
            idx = jnp.min(jnp.where(ws == m, li, big_i), axis=1, keepdims=True)
            pos = l128 == c * 16 + k
            cand_iou = jnp.where(pos, -m[0:_G], cand_iou)
            cand_cv = jnp.where(pos, m[_G:], cand_cv)
            cand_ci = jnp.where(pos, (idx[_G:] + c * _CB).astype(f32), cand_ci)
            ws = jnp.where(li == idx, inf, ws)
        cand_iou_s[...] = cand_iou
        cand_cv_s[...] = cand_cv
        cand_ci_s[...] = cand_ci

    @pl.when(jnp.logical_and(p == 0, c == _NC - 1))
    def _phase_b():
        ic = cand_iou_s[...]
        tsum = jnp.zeros((_G, 1), f32)
        for _ in range(_K):
            m = jnp.max(ic, axis=1, keepdims=True)
            idx = jnp.min(jnp.where(ic == m, l128, big_i), axis=1, keepdims=True)
            tsum = tsum + m
            ic = jnp.where(l128 == idx, -2.0, ic)
        dks = jnp.maximum(jnp.floor(tsum), 1.0)

        cv = cand_cv_s[...]
        ci = cand_ci_s[...]
        sel = jnp.full((_G, 128), -1.0, f32)
        for k in range(_K):
            m = jnp.min(cv, axis=1, keepdims=True)
            idx = jnp.min(jnp.where(cv == m, l128, big_i), axis=1, keepdims=True)
            gidx = jnp.sum(jnp.where(l128 == idx, ci, 0.0), axis=1, keepdims=True)
            take = (m < 1e30) & (jnp.float32(k) < dks)
            sel = jnp.where(l128 == k, jnp.where(take, gidx, -1.0), sel)
            cv = jnp.where(l128 == idx, inf, cv)
        sel_s[...] = sel

    @pl.when(p == 1)
    def _phase_d():
        ot5 = ot5_s[:, sl]
        oo = ot5[4:5, :]
        px = (ot5[0:1, :] + xs) * ss
        py = (ot5[1:2, :] + ys) * ss
        pw = jnp.exp(ot5[2:3, :]) * ss
        ph = jnp.exp(ot5[3:4, :]) * ss

        cost = cost_s[:, sl]
        iou = iou_s[:, sl]
        craw = craw_s[:, sl]
        spsum = sps_s[:, sl]
        sel = sel_s[...]
        lg = (li + c * _CB).astype(f32)

        matching = jnp.zeros((_G, _CB), f32)
        for k in range(_K):
            matching = matching + (lg == sel[:, k:k + 1]).astype(f32)

        col = jnp.sum(matching, axis=0, keepdims=True)
        gi = jax.lax.broadcasted_iota(jnp.int32, (_G, _CB), 0)
        m0 = jnp.min(cost, axis=0, keepdims=True)
        gmin = jnp.min(jnp.where(cost == m0, gi, jnp.int32(_G)), axis=0, keepdims=True)
        gt_oh = (gi == gmin).astype(f32)
        matching = jnp.where(col > 1.0, gt_oh, matching)

        fgf = (jnp.sum(matching, axis=0, keepdims=True) > 0.0).astype(f32)
        miou = jnp.sum(matching * iou, axis=0, keepdims=True)
        nfg = jnp.sum(fgf)

        tb_t = _dotg(lb[:, 0:4], matching, (((0,), (0,))))
        tx, ty, tw, th = tb_t[0:1], tb_t[1:2], tb_t[2:3], tb_t[3:4]
        tlx = jnp.maximum(px - pw / 2, tx - tw / 2)
        tly = jnp.maximum(py - ph / 2, ty - th / 2)
        brx = jnp.minimum(px + pw / 2, tx + tw / 2)
        bry = jnp.minimum(py + ph / 2, ty + th / 2)
        inter = jnp.maximum(brx - tlx, 0.0) * jnp.maximum(bry - tly, 0.0)
        iou_e = inter / (pw * ph + tw * th - inter + 1e-16)
        l_iou = jnp.sum(fgf * (1.0 - iou_e * iou_e))

        xsel = jnp.sum(matching * craw, axis=0, keepdims=True)
        l_cls = jnp.sum(fgf * spsum) - jnp.sum(miou * xsel)

        acc_s[0] = acc_s[0] + l_iou
        acc_s[1] = acc_s[1] - jnp.sum(oo * fgf)
        acc_s[2] = acc_s[2] + l_cls
        acc_s[3] = acc_s[3] + nfg

    @pl.when(jnp.logical_and(p == 1, c == _NC - 1))
    def _emit():
        out_ref[...] = jnp.concatenate(
            [jnp.full((1, 128), acc_s[0], f32),
             jnp.full((1, 128), acc_s[1], f32),
             jnp.full((1, 128), acc_s[2], f32),
             jnp.full((1, 128), acc_s[3], f32)], axis=0)[None]


def kernel(outputs, labels, x_shifts, y_shifts, expanded_strides):
    bs, a, _ = outputs.shape
    pad = _AP - a
    f32 = jnp.float32
    xs = jnp.pad(x_shifts, ((0, 0), (0, pad)), constant_values=-1e4)
    ys = jnp.pad(y_shifts, ((0, 0), (0, pad)), constant_values=-1e4)
    ss = jnp.pad(expanded_strides, ((0, 0), (0, pad)), constant_values=1.0)
    import functools
    kfn = functools.partial(_yolox_kernel, n_valid=a)
    grid_spec = pltpu.PrefetchScalarGridSpec(
        num_scalar_prefetch=0,
        grid=(bs, 2, _NC),
        in_specs=[
            pl.BlockSpec((1, _CB), lambda i, p, c: (0, c)),
            pl.BlockSpec((1, _CB), lambda i, p, c: (0, c)),
            pl.BlockSpec((1, _CB), lambda i, p, c: (0, c)),
            pl.BlockSpec((1, _CB, 5 + _C), lambda i, p, c: (i, c * (1 - p), 0)),
            pl.BlockSpec((1, _G, 5), lambda i, p, c: (i, 0, 0)),
        ],
        out_specs=pl.BlockSpec((1, 4, 128), lambda i, p, c: (i, 0, 0)),
        scratch_shapes=[
            pltpu.VMEM((5, _AP), f32),
            pltpu.VMEM((_G, _AP), f32),
            pltpu.VMEM((_G, _AP), f32),
            pltpu.VMEM((_G, _AP), f32),
            pltpu.VMEM((1, _AP), f32),
            pltpu.VMEM((_G, 128), f32),
            pltpu.VMEM((_G, 128), f32),
            pltpu.VMEM((_G, 128), f32),
            pltpu.VMEM((_G, 128), f32),
            pltpu.SMEM((4,), f32),
        ],
    )
    out = pl.pallas_call(
        kfn,
        grid_spec=grid_spec,
        out_shape=jax.ShapeDtypeStruct((bs, 4, 128), f32),
        compiler_params=pltpu.CompilerParams(
            dimension_semantics=("parallel", "arbitrary", "arbitrary")),
    )(xs, ys, ss, outputs, labels)
    parts = jnp.sum(out[:, :, 0], axis=0)
    return (5.0 * parts[0] + parts[1] + parts[2]) / jnp.maximum(parts[3], 1.0)


# speedup vs baseline: 1.5045x; 1.5045x over previous
import jax
import jax.numpy as jnp
from jax.experimental import pallas as pl
from jax.experimental.pallas import tpu as pltpu

_C = 80
_G = 20
_K = 10
_CB = 4224
_NC = 8
_AP = _CB * _NC
_LOG_EPS = -27.631021


def _dotg(a, b, dims):
    return jax.lax.dot_general(
        a, b, (dims, ((), ())),
        precision=jax.lax.Precision.HIGHEST,
        preferred_element_type=jnp.float32)


def _yolox_kernel(xs_ref, ys_ref, ss_ref, raw_ref, lab_ref, out_ref,
                  ot5_s, cost_s, iou_s, craw_s, sps_s,
                  cand_iou_s, cand_cv_s, cand_ci_s, sel_s, acc_s,
                  *, n_valid):
    f32 = jnp.float32
    p = pl.program_id(1)
    c = pl.program_id(2)
    inf = jnp.float32(jnp.inf)
    big_i = jnp.int32(2 ** 30)
    li = jax.lax.broadcasted_iota(jnp.int32, (1, _CB), 1)
    l128 = jax.lax.broadcasted_iota(jnp.int32, (1, 128), 1)
    sl = pl.ds(c * _CB, _CB)

    xs = xs_ref[...]
    ys = ys_ref[...]
    ss = ss_ref[...]
    lb = lab_ref[...][0]
    tbx, tby, tbw, tbh = lb[:, 0:1], lb[:, 1:2], lb[:, 2:3], lb[:, 3:4]

    @pl.when(jnp.logical_and(p == 0, c == 0))
    def _init():
        cand_iou_s[...] = jnp.full((_G, 128), -2.0, f32)
        cand_cv_s[...] = jnp.full((_G, 128), inf, f32)
        cand_ci_s[...] = jnp.zeros((_G, 128), f32)
        acc_s[0] = 0.0
        acc_s[1] = 0.0
        acc_s[2] = 0.0
        acc_s[3] = 0.0

    @pl.when(p == 0)
    def _phase_a():
        valid = (li + c * _CB) < n_valid
        tr = jnp.transpose(raw_ref[...][0], (1, 0))
        box = jnp.where(valid, tr[0:4], 0.0)
        oo = jnp.where(valid, tr[4:5], -1e9)
        oc = tr[5:, :]
        ot5_s[:, sl] = jnp.concatenate([box, oo], axis=0)

        px = (box[0:1, :] + xs) * ss
        py = (box[1:2, :] + ys) * ss
        pw = jnp.exp(box[2:3, :]) * ss
        ph = jnp.exp(box[3:4, :]) * ss

        oh = (lb[:, 4:5].astype(jnp.int32) ==
              jax.lax.broadcasted_iota(jnp.int32, (_G, _C), 1)).astype(f32)

        xc = (xs + 0.5) * ss
        yc = (ys + 0.5) * ss
        in_box = (jnp.minimum(jnp.minimum(xc - (tbx - tbw / 2), yc - (tby - tbh / 2)),
                              jnp.minimum((tbx + tbw / 2) - xc, (tby + tbh / 2) - yc)) > 0.0)
        rss = 2.5 * ss
        in_mir = (jnp.minimum(jnp.minimum(xc - (tbx - rss), yc - (tby - rss)),
                              jnp.minimum((tbx + rss) - xc, (tby + rss) - yc)) > 0.0)
        union = (jnp.any(in_box, axis=0, keepdims=True) |
                 jnp.any(in_mir, axis=0, keepdims=True))
        inter_m = in_box & in_mir

        tlx = jnp.maximum(tbx - tbw / 2, px - pw / 2)
        tly = jnp.maximum(tby - tbh / 2, py - ph / 2)
        brx = jnp.minimum(tbx + tbw / 2, px + pw / 2)
        bry = jnp.minimum(tby + tbh / 2, py + ph / 2)
        inter = jnp.maximum(brx - tlx, 0.0) * jnp.maximum(bry - tly, 0.0)
        iou = inter / (tbw * tbh + pw * ph - inter + 1e-16)
        iou_s[:, sl] = iou

        eo = jnp.exp(-jnp.abs(oo))
        lo = jnp.log1p(eo)
        lsig_o = jnp.minimum(oo, 0.0) - lo
        e = jnp.exp(-jnp.abs(oc))
        l = jnp.log1p(e)
        a = 0.5 * ((jnp.minimum(oc, 0.0) - l) + lsig_o)
        score = jnp.exp(a)
        l2 = jnp.maximum(jnp.log1p(1e-12 - score), _LOG_EPS)
        s_all = jnp.sum(l2, axis=0, keepdims=True)
        sps_s[:, sl] = jnp.where(
            valid, jnp.sum(jnp.maximum(oc, 0.0) + l, axis=0, keepdims=True), 0.0)
        acc_s[1] = acc_s[1] + jnp.sum(jnp.maximum(oo, 0.0) + lo)
        craw = _dotg(oh, oc, (((1,), (0,))))
        craw_s[:, sl] = jnp.where(valid, craw, 0.0)
        lt = jnp.log1p(jnp.exp(-jnp.abs(craw)))
        at = 0.5 * ((jnp.minimum(craw, 0.0) - lt) + lsig_o)
        g1 = jnp.maximum(at, _LOG_EPS)
        g2 = jnp.maximum(jnp.log1p(1e-12 - jnp.exp(at)), _LOG_EPS)

        cls_loss = -(s_all + g1 - g2)
        iou_loss = -jnp.log(iou + 1e-8)
        cost = cls_loss + 3.0 * iou_loss + 100000.0 * (1.0 - inter_m.astype(f32))
        cost = jnp.where(union, cost, inf)
        cost_s[:, sl] = cost

        ws = jnp.concatenate([-jnp.where(union, iou, -1.0), cost], axis=0)
        cand_iou = cand_iou_s[...]
        cand_cv = cand_cv_s[...]
        cand_ci = cand_ci_s[...]
        for k in range(_K):
            m = jnp.min(ws, axis=1, keepdims=True)
            idx = jnp.min(jnp.where(ws == m, li, big_i), axis=1, keepdims=True)
            pos = l128 == c * 16 + k
            cand_iou = jnp.where(pos, -m[0:_G], cand_iou)
            cand_cv = jnp.where(pos, m[_G:], cand_cv)
            cand_ci = jnp.where(pos, (idx[_G:] + c * _CB).astype(f32), cand_ci)
            ws = jnp.where(li == idx, inf, ws)
        cand_iou_s[...] = cand_iou
        cand_cv_s[...] = cand_cv
        cand_ci_s[...] = cand_ci

    @pl.when(jnp.logical_and(p == 0, c == _NC - 1))
    def _phase_b():
        ic = cand_iou_s[...]
        tsum = jnp.zeros((_G, 1), f32)
        for _ in range(_K):
            m = jnp.max(ic, axis=1, keepdims=True)
            idx = jnp.min(jnp.where(ic == m, l128, big_i), axis=1, keepdims=True)
            tsum = tsum + m
            ic = jnp.where(l128 == idx, -2.0, ic)
        dks = jnp.maximum(jnp.floor(tsum), 1.0)

        cv = cand_cv_s[...]
        ci = cand_ci_s[...]
        sel = jnp.full((_G, 128), -1.0, f32)
        for k in range(_K):
            m = jnp.min(cv, axis=1, keepdims=True)
            idx = jnp.min(jnp.where(cv == m, l128, big_i), axis=1, keepdims=True)
            gidx = jnp.sum(jnp.where(l128 == idx, ci, 0.0), axis=1, keepdims=True)
            take = (m < 1e30) & (jnp.float32(k) < dks)
            sel = jnp.where(l128 == k, jnp.where(take, gidx, -1.0), sel)
            cv = jnp.where(l128 == idx, inf, cv)
        sel_s[...] = sel

    @pl.when(p == 1)
    def _phase_d():
        ot5 = ot5_s[:, sl]
        oo = ot5[4:5, :]
        px = (ot5[0:1, :] + xs) * ss
        py = (ot5[1:2, :] + ys) * ss
        pw = jnp.exp(ot5[2:3, :]) * ss
        ph = jnp.exp(ot5[3:4, :]) * ss

        cost = cost_s[:, sl]
        iou = iou_s[:, sl]
        craw = craw_s[:, sl]
        spsum = sps_s[:, sl]
        sel = sel_s[...]
        lg = (li + c * _CB).astype(f32)

        matching = jnp.zeros((_G, _CB), f32)
        for k in range(_K):
            matching = matching + (lg == sel[:, k:k + 1]).astype(f32)

        col = jnp.sum(matching, axis=0, keepdims=True)
        gi = jax.lax.broadcasted_iota(jnp.int32, (_G, _CB), 0)
        m0 = jnp.min(cost, axis=0, keepdims=True)
        gmin = jnp.min(jnp.where(cost == m0, gi, jnp.int32(_G)), axis=0, keepdims=True)
        gt_oh = (gi == gmin).astype(f32)
        matching = jnp.where(col > 1.0, gt_oh, matching)

        fgf = (jnp.sum(matching, axis=0, keepdims=True) > 0.0).astype(f32)
        miou = jnp.sum(matching * iou, axis=0, keepdims=True)
        nfg = jnp.sum(fgf)

        tb_t = _dotg(lb[:, 0:4], matching, (((0,), (0,))))
        tx, ty, tw, th = tb_t[0:1], tb_t[1:2], tb_t[2:3], tb_t[3:4]
        tlx = jnp.maximum(px - pw / 2, tx - tw / 2)
        tly = jnp.maximum(py - ph / 2, ty - th / 2)
        brx = jnp.minimum(px + pw / 2, tx + tw / 2)
        bry = jnp.minimum(py + ph / 2, ty + th / 2)
        inter = jnp.maximum(brx - tlx, 0.0) * jnp.maximum(bry - tly, 0.0)
        iou_e = inter / (pw * ph + tw * th - inter + 1e-16)
        l_iou = jnp.sum(fgf * (1.0 - iou_e * iou_e))

        xsel = jnp.sum(matching * craw, axis=0, keepdims=True)
        l_cls = jnp.sum(fgf * spsum) - jnp.sum(miou * xsel)

        acc_s[0] = acc_s[0] + l_iou
        acc_s[1] = acc_s[1] - jnp.sum(oo * fgf)
        acc_s[2] = acc_s[2] + l_cls
        acc_s[3] = acc_s[3] + nfg

    @pl.when(jnp.logical_and(p == 1, c == _NC - 1))
    def _emit():
        out_ref[...] = jnp.concatenate(
            [jnp.full((1, 128), acc_s[0], f32),
             jnp.full((1, 128), acc_s[1], f32),
             jnp.full((1, 128), acc_s[2], f32),
             jnp.full((1, 128), acc_s[3], f32)], axis=0)[None]


def kernel(outputs, labels, x_shifts, y_shifts, expanded_strides):
    bs, a, _ = outputs.shape
    pad = _AP - a
    f32 = jnp.float32
    xs = jnp.pad(x_shifts, ((0, 0), (0, pad)), constant_values=-1e4)
    ys = jnp.pad(y_shifts, ((0, 0), (0, pad)), constant_values=-1e4)
    ss = jnp.pad(expanded_strides, ((0, 0), (0, pad)), constant_values=1.0)
    import functools
    kfn = functools.partial(_yolox_kernel, n_valid=a)
    grid_spec = pltpu.PrefetchScalarGridSpec(
        num_scalar_prefetch=0,
        grid=(bs, 2, _NC),
        in_specs=[
            pl.BlockSpec((1, _CB), lambda i, p, c: (0, c)),
            pl.BlockSpec((1, _CB), lambda i, p, c: (0, c)),
            pl.BlockSpec((1, _CB), lambda i, p, c: (0, c)),
            pl.BlockSpec((1, _CB, 5 + _C), lambda i, p, c: (i, c * (1 - p), 0)),
            pl.BlockSpec((1, _G, 5), lambda i, p, c: (i, 0, 0)),
        ],
        out_specs=pl.BlockSpec((1, 4, 128), lambda i, p, c: (i, 0, 0)),
        scratch_shapes=[
            pltpu.VMEM((5, _AP), f32),
            pltpu.VMEM((_G, _AP), f32),
            pltpu.VMEM((_G, _AP), f32),
            pltpu.VMEM((_G, _AP), f32),
            pltpu.VMEM((1, _AP), f32),
            pltpu.VMEM((_G, 128), f32),
            pltpu.VMEM((_G, 128), f32),
            pltpu.VMEM((_G, 128), f32),
            pltpu.VMEM((_G, 128), f32),
            pltpu.SMEM((4,), f32),
        ],
    )
    out = pl.pallas_call(
        kfn,
        grid_spec=grid_spec,
        out_shape=jax.ShapeDtypeStruct((bs, 4, 128), f32),
        compiler_params=pltpu.CompilerParams(
            dimension_semantics=("parallel", "arbitrary", "arbitrary")),
    )(xs, ys, ss, outputs, labels)
    parts = jnp.sum(out[:, :, 0], axis=0)
    return (5.0 * parts[0] + parts[1] + parts[2]) / jnp.maximum(parts[3], 1.0)


# speedup vs baseline: 1.6518x; 1.0979x over previous
import jax
import jax.numpy as jnp
from jax.experimental import pallas as pl
from jax.experimental.pallas import tpu as pltpu

_C = 80
_G = 20
_K = 10
_CB = 8448
_NC = 4
_AP = _CB * _NC
_LOG_EPS = -27.631021


def _dotg(a, b, dims):
    return jax.lax.dot_general(
        a, b, (dims, ((), ())),
        precision=jax.lax.Precision.HIGHEST,
        preferred_element_type=jnp.float32)


def _yolox_kernel(xs_ref, ys_ref, ss_ref, raw_ref, lab_ref, out_ref,
                  ot5_s, cost_s, iou_s, craw_s, sps_s,
                  cand_iou_s, cand_cv_s, cand_ci_s, sel_s, acc_s,
                  *, n_valid):
    f32 = jnp.float32
    p = pl.program_id(1)
    c = pl.program_id(2)
    inf = jnp.float32(jnp.inf)
    big_i = jnp.int32(2 ** 30)
    li = jax.lax.broadcasted_iota(jnp.int32, (1, _CB), 1)
    l128 = jax.lax.broadcasted_iota(jnp.int32, (1, 128), 1)
    sl = pl.ds(c * _CB, _CB)

    xs = xs_ref[...]
    ys = ys_ref[...]
    ss = ss_ref[...]
    lb = lab_ref[...][0]
    tbx, tby, tbw, tbh = lb[:, 0:1], lb[:, 1:2], lb[:, 2:3], lb[:, 3:4]

    @pl.when(jnp.logical_and(p == 0, c == 0))
    def _init():
        cand_iou_s[...] = jnp.full((_G, 128), -2.0, f32)
        cand_cv_s[...] = jnp.full((_G, 128), inf, f32)
        cand_ci_s[...] = jnp.zeros((_G, 128), f32)
        acc_s[0] = 0.0
        acc_s[1] = 0.0
        acc_s[2] = 0.0
        acc_s[3] = 0.0

    @pl.when(p == 0)
    def _phase_a():
        valid = (li + c * _CB) < n_valid
        tr = jnp.transpose(raw_ref[...][0], (1, 0))
        box = jnp.where(valid, tr[0:4], 0.0)
        oo = jnp.where(valid, tr[4:5], -1e9)
        oc = tr[5:, :]
        ot5_s[:, sl] = jnp.concatenate([box, oo], axis=0)

        px = (box[0:1, :] + xs) * ss
        py = (box[1:2, :] + ys) * ss
        pw = jnp.exp(box[2:3, :]) * ss
        ph = jnp.exp(box[3:4, :]) * ss

        oh = (lb[:, 4:5].astype(jnp.int32) ==
              jax.lax.broadcasted_iota(jnp.int32, (_G, _C), 1)).astype(f32)

        xc = (xs + 0.5) * ss
        yc = (ys + 0.5) * ss
        in_box = (jnp.minimum(jnp.minimum(xc - (tbx - tbw / 2), yc - (tby - tbh / 2)),
                              jnp.minimum((tbx + tbw / 2) - xc, (tby + tbh / 2) - yc)) > 0.0)
        rss = 2.5 * ss
        in_mir = (jnp.minimum(jnp.minimum(xc - (tbx - rss), yc - (tby - rss)),
                              jnp.minimum((tbx + rss) - xc, (tby + rss) - yc)) > 0.0)
        union = (jnp.any(in_box, axis=0, keepdims=True) |
                 jnp.any(in_mir, axis=0, keepdims=True))
        inter_m = in_box & in_mir

        tlx = jnp.maximum(tbx - tbw / 2, px - pw / 2)
        tly = jnp.maximum(tby - tbh / 2, py - ph / 2)
        brx = jnp.minimum(tbx + tbw / 2, px + pw / 2)
        bry = jnp.minimum(tby + tbh / 2, py + ph / 2)
        inter = jnp.maximum(brx - tlx, 0.0) * jnp.maximum(bry - tly, 0.0)
        iou = inter / (tbw * tbh + pw * ph - inter + 1e-16)
        iou_s[:, sl] = iou

        eo = jnp.exp(-jnp.abs(oo))
        lo = jnp.log1p(eo)
        lsig_o = jnp.minimum(oo, 0.0) - lo
        e = jnp.exp(-jnp.abs(oc))
        l = jnp.log1p(e)
        a = 0.5 * ((jnp.minimum(oc, 0.0) - l) + lsig_o)
        score = jnp.exp(a)
        l2 = jnp.maximum(jnp.log1p(1e-12 - score), _LOG_EPS)
        s_all = jnp.sum(l2, axis=0, keepdims=True)
        sps_s[:, sl] = jnp.where(
            valid, jnp.sum(jnp.maximum(oc, 0.0) + l, axis=0, keepdims=True), 0.0)
        acc_s[1] = acc_s[1] + jnp.sum(jnp.maximum(oo, 0.0) + lo)
        craw = _dotg(oh, oc, (((1,), (0,))))
        craw_s[:, sl] = jnp.where(valid, craw, 0.0)
        lt = jnp.log1p(jnp.exp(-jnp.abs(craw)))
        at = 0.5 * ((jnp.minimum(craw, 0.0) - lt) + lsig_o)
        g1 = jnp.maximum(at, _LOG_EPS)
        g2 = jnp.maximum(jnp.log1p(1e-12 - jnp.exp(at)), _LOG_EPS)

        cls_loss = -(s_all + g1 - g2)
        iou_loss = -jnp.log(iou + 1e-8)
        cost = cls_loss + 3.0 * iou_loss + 100000.0 * (1.0 - inter_m.astype(f32))
        cost = jnp.where(union, cost, inf)
        cost_s[:, sl] = cost

        ws = jnp.concatenate([-jnp.where(union, iou, -1.0), cost], axis=0)
        cand_iou = cand_iou_s[...]
        cand_cv = cand_cv_s[...]
        cand_ci = cand_ci_s[...]
        for k in range(_K):
            m = jnp.min(ws, axis=1, keepdims=True)
            idx = jnp.min(jnp.where(ws == m, li, big_i), axis=1, keepdims=True)
            pos = l128 == c * 16 + k
            cand_iou = jnp.where(pos, -m[0:_G], cand_iou)
            cand_cv = jnp.where(pos, m[_G:], cand_cv)
            cand_ci = jnp.where(pos, (idx[_G:] + c * _CB).astype(f32), cand_ci)
            ws = jnp.where(li == idx, inf, ws)
        cand_iou_s[...] = cand_iou
        cand_cv_s[...] = cand_cv
        cand_ci_s[...] = cand_ci

    @pl.when(jnp.logical_and(p == 0, c == _NC - 1))
    def _phase_b():
        ic = cand_iou_s[...]
        tsum = jnp.zeros((_G, 1), f32)
        for _ in range(_K):
            m = jnp.max(ic, axis=1, keepdims=True)
            idx = jnp.min(jnp.where(ic == m, l128, big_i), axis=1, keepdims=True)
            tsum = tsum + m
            ic = jnp.where(l128 == idx, -2.0, ic)
        dks = jnp.maximum(jnp.floor(tsum), 1.0)

        cv = cand_cv_s[...]
        ci = cand_ci_s[...]
        sel = jnp.full((_G, 128), -1.0, f32)
        for k in range(_K):
            m = jnp.min(cv, axis=1, keepdims=True)
            idx = jnp.min(jnp.where(cv == m, l128, big_i), axis=1, keepdims=True)
            gidx = jnp.sum(jnp.where(l128 == idx, ci, 0.0), axis=1, keepdims=True)
            take = (m < 1e30) & (jnp.float32(k) < dks)
            sel = jnp.where(l128 == k, jnp.where(take, gidx, -1.0), sel)
            cv = jnp.where(l128 == idx, inf, cv)
        sel_s[...] = sel

    @pl.when(p == 1)
    def _phase_d():
        ot5 = ot5_s[:, sl]
        oo = ot5[4:5, :]
        px = (ot5[0:1, :] + xs) * ss
        py = (ot5[1:2, :] + ys) * ss
        pw = jnp.exp(ot5[2:3, :]) * ss
        ph = jnp.exp(ot5[3:4, :]) * ss

        cost = cost_s[:, sl]
        iou = iou_s[:, sl]
        craw = craw_s[:, sl]
        spsum = sps_s[:, sl]
        sel = sel_s[...]
        lg = (li + c * _CB).astype(f32)

        matching = jnp.zeros((_G, _CB), f32)
        for k in range(_K):
            matching = matching + (lg == sel[:, k:k + 1]).astype(f32)

        col = jnp.sum(matching, axis=0, keepdims=True)
        gi = jax.lax.broadcasted_iota(jnp.int32, (_G, _CB), 0)
        m0 = jnp.min(cost, axis=0, keepdims=True)
        gmin = jnp.min(jnp.where(cost == m0, gi, jnp.int32(_G)), axis=0, keepdims=True)
        gt_oh = (gi == gmin).astype(f32)
        matching = jnp.where(col > 1.0, gt_oh, matching)

        fgf = (jnp.sum(matching, axis=0, keepdims=True) > 0.0).astype(f32)
        miou = jnp.sum(matching * iou, axis=0, keepdims=True)
        nfg = jnp.sum(fgf)

        tb_t = _dotg(lb[:, 0:4], matching, (((0,), (0,))))
        tx, ty, tw, th = tb_t[0:1], tb_t[1:2], tb_t[2:3], tb_t[3:4]
        tlx = jnp.maximum(px - pw / 2, tx - tw / 2)
        tly = jnp.maximum(py - ph / 2, ty - th / 2)
        brx = jnp.minimum(px + pw / 2, tx + tw / 2)
        bry = jnp.minimum(py + ph / 2, ty + th / 2)
        inter = jnp.maximum(brx - tlx, 0.0) * jnp.maximum(bry - tly, 0.0)
        iou_e = inter / (pw * ph + tw * th - inter + 1e-16)
        l_iou = jnp.sum(fgf * (1.0 - iou_e * iou_e))

        xsel = jnp.sum(matching * craw, axis=0, keepdims=True)
        l_cls = jnp.sum(fgf * spsum) - jnp.sum(miou * xsel)

        acc_s[0] = acc_s[0] + l_iou
        acc_s[1] = acc_s[1] - jnp.sum(oo * fgf)
        acc_s[2] = acc_s[2] + l_cls
        acc_s[3] = acc_s[3] + nfg

    @pl.when(jnp.logical_and(p == 1, c == _NC - 1))
    def _emit():
        out_ref[...] = jnp.concatenate(
            [jnp.full((1, 128), acc_s[0], f32),
             jnp.full((1, 128), acc_s[1], f32),
             jnp.full((1, 128), acc_s[2], f32),
             jnp.full((1, 128), acc_s[3], f32)], axis=0)[None]


def kernel(outputs, labels, x_shifts, y_shifts, expanded_strides):
    bs, a, _ = outputs.shape
    pad = _AP - a
    f32 = jnp.float32
    xs = jnp.pad(x_shifts, ((0, 0), (0, pad)), constant_values=-1e4)
    ys = jnp.pad(y_shifts, ((0, 0), (0, pad)), constant_values=-1e4)
    ss = jnp.pad(expanded_strides, ((0, 0), (0, pad)), constant_values=1.0)
    import functools
    kfn = functools.partial(_yolox_kernel, n_valid=a)
    grid_spec = pltpu.PrefetchScalarGridSpec(
        num_scalar_prefetch=0,
        grid=(bs, 2, _NC),
        in_specs=[
            pl.BlockSpec((1, _CB), lambda i, p, c: (0, c)),
            pl.BlockSpec((1, _CB), lambda i, p, c: (0, c)),
            pl.BlockSpec((1, _CB), lambda i, p, c: (0, c)),
            pl.BlockSpec((1, _CB, 5 + _C), lambda i, p, c: (i, c * (1 - p), 0)),
            pl.BlockSpec((1, _G, 5), lambda i, p, c: (i, 0, 0)),
        ],
        out_specs=pl.BlockSpec((1, 4, 128), lambda i, p, c: (i, 0, 0)),
        scratch_shapes=[
            pltpu.VMEM((5, _AP), f32),
            pltpu.VMEM((_G, _AP), f32),
            pltpu.VMEM((_G, _AP), f32),
            pltpu.VMEM((_G, _AP), f32),
            pltpu.VMEM((1, _AP), f32),
            pltpu.VMEM((_G, 128), f32),
            pltpu.VMEM((_G, 128), f32),
            pltpu.VMEM((_G, 128), f32),
            pltpu.VMEM((_G, 128), f32),
            pltpu.SMEM((4,), f32),
        ],
    )
    out = pl.pallas_call(
        kfn,
        grid_spec=grid_spec,
        out_shape=jax.ShapeDtypeStruct((bs, 4, 128), f32),
        compiler_params=pltpu.CompilerParams(
            dimension_semantics=("parallel", "arbitrary", "arbitrary")),
    )(xs, ys, ss, outputs, labels)
    parts = jnp.sum(out[:, :, 0], axis=0)
    return (5.0 * parts[0] + parts[1] + parts[2]) / jnp.maximum(parts[3], 1.0)
